# baseline (speedup 1.0000x reference)
import jax
import jax.numpy as jnp
from jax.experimental import pallas as pl

N = 50000
NT = 50000
B = 500


def _gcn_conv(h_in, src, dst, W, b, n):
    h = h_in @ W
    loop = jnp.arange(n, dtype=src.dtype)
    s = jnp.concatenate([src, loop])
    d = jnp.concatenate([dst, loop])
    deg = jnp.zeros((n,), jnp.float32).at[d].add(1.0)
    dinv = jnp.where(deg > 0, jax.lax.rsqrt(jnp.maximum(deg, 1e-12)), 0.0)
    norm = dinv[s] * dinv[d]
    msg = h[s] * norm[:, None]
    out = jnp.zeros((n, W.shape[1]), jnp.float32).at[d].add(msg)
    return out + b


def _head_matmul_kernel(x_ref, w_ref, b_ref, o_ref):
    o_ref[...] = x_ref[...] @ w_ref[...] + b_ref[...]


def _pallas_linear(x, w, b):
    m, k = x.shape
    n = w.shape[1]
    return pl.pallas_call(
        _head_matmul_kernel,
        out_shape=jax.ShapeDtypeStruct((m, n), jnp.float32),
    )(x, w, b[None, :])


def kernel(x, edge_index, batch, xt, edge_index_t, batch_t, params):
    relu = jax.nn.relu
    p = params
    s, d = edge_index[0], edge_index[1]
    d, s = jax.lax.sort((d, s), num_keys=1)
    st_, dt_ = edge_index_t[0], edge_index_t[1]
    dt_, st_ = jax.lax.sort((dt_, st_), num_keys=1)
    row_starts = jnp.searchsorted(d, jnp.arange(0, 53248 + 1, 128, dtype=jnp.int32))
    row_starts_t = jnp.searchsorted(dt_, jnp.arange(0, 53248 + 1, 128, dtype=jnp.int32))
    s = s + jnp.min(row_starts).astype(s.dtype) * 0
    st_ = st_ + jnp.min(row_starts_t).astype(st_.dtype) * 0
    h = relu(_gcn_conv(x, s, d, p['W1'], p['b1'], N))
    h = relu(_gcn_conv(h, s, d, p['W2'], p['b2'], N))
    h = relu(_gcn_conv(h, s, d, p['W3'], p['b3'], N))
    h = relu(_gcn_conv(h, s, d, p['W4'], p['b4'], N))
    g = jax.ops.segment_max(h, batch, num_segments=B)
    g = relu(_pallas_linear(g, p['fg1W'], p['fg1b']))
    g = _pallas_linear(g, p['fg2W'], p['fg2b'])

    st, dt = st_, dt_
    ht = relu(_gcn_conv(xt, st, dt, p['Wt1'], p['bt1'], NT))
    ht = relu(_gcn_conv(ht, st, dt, p['Wt2'], p['bt2'], NT))
    ht = relu(_gcn_conv(ht, st, dt, p['Wt3'], p['bt3'], NT))
    ht = relu(_gcn_conv(ht, st, dt, p['Wt4'], p['bt4'], NT))
    gt = jax.ops.segment_max(ht, batch_t, num_segments=B)
    gt = relu(_pallas_linear(gt, p['fg1tW'], p['fg1tb']))
    gt = _pallas_linear(gt, p['fg2tW'], p['fg2tb'])

    xc = jnp.concatenate([g, gt], axis=1)
    xc = relu(_pallas_linear(xc, p['fc1W'], p['fc1b']))
    xc = relu(_pallas_linear(xc, p['fc2W'], p['fc2b']))
    return _pallas_linear(xc, p['outW'], p['outb'])


# speedup vs baseline: 3.6858x; 3.6858x over previous
import functools

import jax
import jax.numpy as jnp
from jax import lax
from jax.experimental import pallas as pl
from jax.experimental.pallas import tpu as pltpu
from jax.experimental.pallas import tpu_sc as plsc

N = 50000
E = 800000
BSEG = 500
BLKN = 512
NBLK = 98
NPAD = NBLK * BLKN
NZV = NPAD // 16
CE = 8000
NCH = E // CE
NW = 32

@functools.lru_cache(maxsize=None)
def _get_mesh():
    return plsc.VectorSubcoreMesh(core_axis_name="c", subcore_axis_name="s")


_SC_PARAMS = pltpu.CompilerParams(needs_layout_passes=False)

_F32 = jnp.float32
_I32 = jnp.int32


def _zero_vmem(ref, nvec):
    z = jnp.zeros((16,), _F32)

    def body(i, _):
        ref[pl.ds(i * 16, 16)] = z
        return 0

    lax.fori_loop(0, nvec, body, 0, unroll=8)


@functools.lru_cache(maxsize=None)
def _get_deg_kernel():
    return functools.partial(
        pl.kernel,
        out_type=(jax.ShapeDtypeStruct((NPAD,), _F32),
                  jax.ShapeDtypeStruct((NPAD,), _F32)),
        mesh=_get_mesh(),
        scratch_types=[pltpu.VMEM((NPAD,), _F32),
                       pltpu.VMEM((2000,), _I32),
                       pltpu.VMEM((NPAD // 32,), _F32),
                       pltpu.VMEM((NPAD // 32,), _F32),
                       pltpu.VMEM_SHARED((16 * NPAD,), _F32)],
        compiler_params=_SC_PARAMS,
    )(_deg_body)


def _deg_body(epd, ept, cnt_d, cnt_t, acc_v, ids_v, tmp_v, sum_v, shared):
    c = lax.axis_index("c")
    s = lax.axis_index("s")
    ones = jnp.full((16,), 1.0, _F32)
    epr = E // 16
    colw = NPAD // 32

    def count_into_acc(ep_ref):
        _zero_vmem(acc_v, NZV)

        def chunk(cc, _):
            pltpu.sync_copy(ep_ref.at[pl.ds(s * epr + cc * 2000, 2000)], ids_v)

            def vec(i, _):
                u = ids_v[pl.ds(i * 16, 16)]
                d = jnp.bitwise_and(u >> 16, 0xFFFF)
                plsc.addupdate_scatter(acc_v, [d], ones)
                return 0

            lax.fori_loop(0, 125, vec, 0, unroll=5)
            return 0

        lax.fori_loop(0, 25, chunk, 0)

    @pl.when(c == 0)
    def _():
        count_into_acc(epd)

    @pl.when(c == 1)
    def _():
        count_into_acc(ept)

    pltpu.sync_copy(acc_v, shared.at[pl.ds(s * NPAD, NPAD)])
    plsc.subcore_barrier()

    col0 = s * (NPAD // 16)

    def reduce_half(h, out_ref):
        base = col0 + h * colw
        _zero_vmem(sum_v, colw // 16)

        def row(r, _):
            pltpu.sync_copy(shared.at[pl.ds(r * NPAD + base, colw)], tmp_v)

            def add(j, _):
                sl = pl.ds(j * 16, 16)
                sum_v[sl] = sum_v[sl] + tmp_v[sl]
                return 0

            lax.fori_loop(0, colw // 16, add, 0, unroll=4)
            return 0

        lax.fori_loop(0, 16, row, 0)
        pltpu.sync_copy(sum_v, out_ref.at[pl.ds(base, colw)])

    @pl.when(c == 0)
    def _():
        reduce_half(0, cnt_d)
        reduce_half(1, cnt_d)

    @pl.when(c == 1)
    def _():
        reduce_half(0, cnt_t)
        reduce_half(1, cnt_t)


@functools.lru_cache(maxsize=None)
def _make_agg(fd, ft):
    npass = -(-(fd + ft) // NW)

    @functools.partial(
        pl.kernel,
        out_type=(jax.ShapeDtypeStruct((fd, NPAD), _F32),
                  jax.ShapeDtypeStruct((ft, NPAD), _F32)),
        mesh=_get_mesh(),
        scratch_types=[pltpu.VMEM((NPAD,), _F32),
                       pltpu.VMEM((NPAD,), _F32),
                       pltpu.VMEM((CE,), _I32)],
        compiler_params=_SC_PARAMS,
    )
    def agg(yd, yt, epd, ept, sd, st, y_v, out_v, ids_v):
        c = lax.axis_index("c")
        s = lax.axis_index("s")
        wid = s + 16 * c

        def do_feature(y_ref, ep_ref, out_hbm, f):
            pltpu.sync_copy(y_ref.at[f], y_v)
            _zero_vmem(out_v, NZV)

            def chunk(cc, _):
                pltpu.sync_copy(ep_ref.at[pl.ds(cc * CE, CE)], ids_v)

                def vec(i, _):
                    u = ids_v[pl.ds(i * 16, 16)]
                    d = jnp.bitwise_and(u >> 16, 0xFFFF)
                    src = jnp.bitwise_and(u, 0xFFFF)
                    g = plsc.load_gather(y_v, [src])
                    plsc.addupdate_scatter(out_v, [d], g)
                    return 0

                lax.fori_loop(0, CE // 16, vec, 0, unroll=4)
                return 0

            lax.fori_loop(0, NCH, chunk, 0)
            pltpu.sync_copy(out_v, out_hbm.at[f])

        def passes(p, _):
            fid = p * NW + wid

            @pl.when(fid < fd)
            def _():
                do_feature(yd, epd, sd, fid)

            @pl.when(jnp.logical_and(fid >= fd, fid < fd + ft))
            def _():
                do_feature(yt, ept, st, fid - fd)

            return 0

        lax.fori_loop(0, npass, passes, 0)

    return agg


@functools.lru_cache(maxsize=None)
def _get_pool_kernel():
    return functools.partial(
        pl.kernel,
        out_type=(jax.ShapeDtypeStruct((320, BLKN), _F32),
                  jax.ShapeDtypeStruct((128, BLKN), _F32)),
        mesh=_get_mesh(),
        scratch_types=[pltpu.VMEM((NPAD,), _F32),
                       pltpu.VMEM((BLKN,), _F32),
                       pltpu.VMEM((528,), _I32),
                       pltpu.VMEM((528,), _I32),
                       pltpu.VMEM((528,), _I32),
                       pltpu.VMEM((528,), _I32)],
        compiler_params=_SC_PARAMS,
    )(_pool_body)


def _pool_body(hd, ht, ssd, sed, sst, set_, gd, gt,
               h_v, g_v, sa_d, se_d, sa_t, se_t):
    c = lax.axis_index("c")
    s = lax.axis_index("s")
    wid = s + 16 * c
    pltpu.sync_copy(ssd, sa_d)
    pltpu.sync_copy(sed, se_d)
    pltpu.sync_copy(sst, sa_t)
    pltpu.sync_copy(set_, se_t)
    lanes = lax.iota(_I32, 16)
    ninf = jnp.full((16,), -jnp.inf, _F32)

    def do_feature(h_ref, sa_v, se_v, g_ref, f):
        pltpu.sync_copy(h_ref.at[f], h_v)

        def grp(g, _):
            sv = sa_v[pl.ds(g * 16, 16)]
            ev = se_v[pl.ds(g * 16, 16)]
            res = ninf
            for j in range(16):
                a = jnp.sum(jnp.where(lanes == j, sv, 0))
                b = jnp.sum(jnp.where(lanes == j, ev, 0))
                a0 = jnp.bitwise_and(a, -16)
                nt = (b - a0 + 15) >> 4

                def scanrows(k, m):
                    base = a0 + k * 16
                    v = h_v[pl.ds(base, 16)]
                    lane = base + lanes
                    ok = jnp.logical_and(lane >= a, lane < b)
                    return jnp.maximum(m, jnp.where(ok, v, ninf))

                m = lax.fori_loop(0, nt, scanrows, ninf)
                gmax = jnp.max(m)
                res = jnp.where(lanes == j, gmax, res)
            g_v[pl.ds(g * 16, 16)] = res
            return 0

        lax.fori_loop(0, BLKN // 16, grp, 0)
        pltpu.sync_copy(g_v, g_ref.at[f])

    def passes(p, _):
        fid = p * NW + wid

        @pl.when(fid < 320)
        def _():
            do_feature(hd, sa_d, se_d, gd, fid)

        @pl.when(jnp.logical_and(fid >= 320, fid < 448))
        def _():
            do_feature(ht, sa_t, se_t, gt, fid - 320)

        return 0

    lax.fori_loop(0, 14, passes, 0)


def _k0(xp, cnt_row, fp):
    def body(x_ref, c_ref, y_ref, di_ref):
        dinv = lax.rsqrt(c_ref[...] + 1.0)
        xt = jnp.transpose(x_ref[...])
        y_ref[...] = xt * dinv
        di_ref[...] = dinv

    return pl.pallas_call(
        body,
        grid=(NBLK,),
        in_specs=[pl.BlockSpec((BLKN, fp), lambda n: (n, 0)),
                  pl.BlockSpec((1, BLKN), lambda n: (0, n))],
        out_specs=[pl.BlockSpec((fp, BLKN), lambda n: (0, n)),
                   pl.BlockSpec((1, BLKN), lambda n: (0, n))],
        out_shape=[jax.ShapeDtypeStruct((fp, NPAD), _F32),
                   jax.ShapeDtypeStruct((1, NPAD), _F32)],
    )(xp, cnt_row)


def _layer(s_t, y_t, dinv, w_t, b_col, final_scale):
    fout, fin = w_t.shape

    def body(s_ref, y_ref, d_ref, w_ref, b_ref, o_ref):
        dv = d_ref[...]
        z = dv * (s_ref[...] + y_ref[...])
        h = jnp.dot(w_ref[...], z, preferred_element_type=_F32, precision=lax.Precision.HIGHEST) + b_ref[...]
        h = jnp.maximum(h, 0.0)
        o_ref[...] = h * dv if final_scale else h

    return pl.pallas_call(
        body,
        grid=(NBLK,),
        in_specs=[pl.BlockSpec((fin, BLKN), lambda n: (0, n)),
                  pl.BlockSpec((fin, BLKN), lambda n: (0, n)),
                  pl.BlockSpec((1, BLKN), lambda n: (0, n)),
                  pl.BlockSpec((fout, fin), lambda n: (0, 0)),
                  pl.BlockSpec((fout, 1), lambda n: (0, 0))],
        out_specs=pl.BlockSpec((fout, BLKN), lambda n: (0, n)),
        out_shape=jax.ShapeDtypeStruct((fout, NPAD), _F32),
    )(s_t, y_t, dinv, w_t, b_col)


def _head(gd, gt, w):
    def body(gd_ref, gt_ref, w1d, b1d, w2d, b2d, w1t, b1t, w2t, b2t,
             wca, wcb, bc, w2, b2, wo, bo, o_ref):
        g1d = jnp.maximum(jnp.dot(w1d[...], gd_ref[...],
                                  preferred_element_type=_F32, precision=lax.Precision.HIGHEST) + b1d[...], 0.0)
        g2d = jnp.dot(w2d[...], g1d, preferred_element_type=_F32, precision=lax.Precision.HIGHEST) + b2d[...]
        g1t = jnp.maximum(jnp.dot(w1t[...], gt_ref[...],
                                  preferred_element_type=_F32, precision=lax.Precision.HIGHEST) + b1t[...], 0.0)
        g2t = jnp.dot(w2t[...], g1t, preferred_element_type=_F32, precision=lax.Precision.HIGHEST) + b2t[...]
        c1 = jnp.maximum(jnp.dot(wca[...], g2d, preferred_element_type=_F32, precision=lax.Precision.HIGHEST)
                         + jnp.dot(wcb[...], g2t, preferred_element_type=_F32, precision=lax.Precision.HIGHEST)
                         + bc[...], 0.0)
        c2 = jnp.maximum(jnp.dot(w2[...], c1, preferred_element_type=_F32, precision=lax.Precision.HIGHEST)
                         + b2[...], 0.0)
        o_ref[...] = jnp.dot(wo[...], c2, preferred_element_type=_F32, precision=lax.Precision.HIGHEST) + bo[...]

    full = lambda a: pl.BlockSpec(a.shape, lambda: tuple(0 for _ in a.shape))
    args = [gd, gt] + w
    return pl.pallas_call(
        body,
        in_specs=[full(a) for a in args],
        out_specs=pl.BlockSpec((1, BLKN), lambda: (0, 0)),
        out_shape=jax.ShapeDtypeStruct((1, BLKN), _F32),
    )(*args)


def _pad_wt(wmat, fin_p, fout_p):
    wt = wmat.T
    return jnp.zeros((fout_p, fin_p), _F32).at[:wt.shape[0], :wt.shape[1]].set(wt)


def _pad_b(b, fp):
    return jnp.zeros((fp, 1), _F32).at[:b.shape[0], 0].set(b)


def kernel(x, edge_index, batch, xt, edge_index_t, batch_t, params):
    p = params

    xp = jnp.zeros((NPAD, 80), _F32).at[:N, :78].set(x)
    xtp = jnp.zeros((NPAD, 32), _F32).at[:N, :30].set(xt)
    epd = jnp.bitwise_or(edge_index[1] << 16, edge_index[0])
    ept = jnp.bitwise_or(edge_index_t[1] << 16, edge_index_t[0])

    segs = jnp.arange(BSEG + 1, dtype=_I32)
    ssd_full = jnp.searchsorted(batch, segs).astype(_I32)
    sst_full = jnp.searchsorted(batch_t, segs).astype(_I32)
    padn = jnp.full((27,), N, _I32)
    ssd = jnp.concatenate([ssd_full[:-1], padn, jnp.zeros((1,), _I32)])[:528]
    sed = jnp.concatenate([ssd_full[1:], padn, jnp.zeros((1,), _I32)])[:528]
    sst = jnp.concatenate([sst_full[:-1], padn, jnp.zeros((1,), _I32)])[:528]
    set_ = jnp.concatenate([sst_full[1:], padn, jnp.zeros((1,), _I32)])[:528]

    cnt_d, cnt_t = _get_deg_kernel()(epd, ept)

    y_d, dinv_d = _k0(xp, cnt_d.reshape(1, NPAD), 80)
    y_t, dinv_t = _k0(xtp, cnt_t.reshape(1, NPAD), 32)

    wds = [(_pad_wt(p['W1'], 80, 80), _pad_b(p['b1'], 80)),
           (_pad_wt(p['W2'], 80, 160), _pad_b(p['b2'], 160)),
           (_pad_wt(p['W3'], 160, 320), _pad_b(p['b3'], 320)),
           (_pad_wt(p['W4'], 320, 320), _pad_b(p['b4'], 320))]
    wts = [(_pad_wt(p['Wt1'], 32, 32), _pad_b(p['bt1'], 32)),
           (_pad_wt(p['Wt2'], 32, 64), _pad_b(p['bt2'], 64)),
           (_pad_wt(p['Wt3'], 64, 128), _pad_b(p['bt3'], 128)),
           (_pad_wt(p['Wt4'], 128, 128), _pad_b(p['bt4'], 128))]

    aggs = [_make_agg(80, 32), _make_agg(80, 32),
            _make_agg(160, 64), _make_agg(320, 128)]
    for i in range(4):
        s_d, s_t = aggs[i](y_d, y_t, epd, ept)
        fin_scale = i < 3
        y_d = _layer(s_d, y_d, dinv_d, wds[i][0], wds[i][1], fin_scale)
        y_t = _layer(s_t, y_t, dinv_t, wts[i][0], wts[i][1], fin_scale)

    g_d, g_t = _get_pool_kernel()(y_d, y_t, ssd, sed, sst, set_)

    headw = [_pad_wt(p['fg1W'], 320, 1024), _pad_b(p['fg1b'], 1024),
             _pad_wt(p['fg2W'], 1024, 1280), _pad_b(p['fg2b'], 1280),
             _pad_wt(p['fg1tW'], 128, 1024), _pad_b(p['fg1tb'], 1024),
             _pad_wt(p['fg2tW'], 1024, 1280), _pad_b(p['fg2tb'], 1280),
             _pad_wt(p['fc1W'][:1280], 1280, 1024),
             _pad_wt(p['fc1W'][1280:], 1280, 1024), _pad_b(p['fc1b'], 1024),
             _pad_wt(p['fc2W'], 1024, 512), _pad_b(p['fc2b'], 512),
             _pad_wt(p['outW'], 512, 1), _pad_b(p['outb'], 1)]

    out = _head(g_d, g_t, headw)
    return out[0, :BSEG][:, None]


# speedup vs baseline: 8.0796x; 2.1921x over previous
import functools

import jax
import jax.numpy as jnp
from jax import lax
from jax.experimental import pallas as pl
from jax.experimental.pallas import tpu as pltpu
from jax.experimental.pallas import tpu_sc as plsc

N = 50000
E = 800000
BSEG = 500
BLKN = 512
NBLK = 98
NPAD = NBLK * BLKN
NZV = NPAD // 16
CE = 16000
NCH = E // CE
NW = 32

@functools.lru_cache(maxsize=None)
def _get_mesh():
    return plsc.VectorSubcoreMesh(core_axis_name="c", subcore_axis_name="s")


_SC_PARAMS = pltpu.CompilerParams(needs_layout_passes=False)

_F32 = jnp.float32
_I32 = jnp.int32


def _zero_vmem(ref, nvec):
    z = jnp.zeros((16,), _F32)

    def body(i, _):
        ref[pl.ds(i * 16, 16)] = z
        return 0

    lax.fori_loop(0, nvec, body, 0, unroll=8)


@functools.lru_cache(maxsize=None)
def _get_deg_kernel():
    return functools.partial(
        pl.kernel,
        out_type=(jax.ShapeDtypeStruct((NPAD,), _F32),
                  jax.ShapeDtypeStruct((NPAD,), _F32)),
        mesh=_get_mesh(),
        scratch_types=[pltpu.VMEM((NPAD,), _F32),
                       pltpu.VMEM((2000,), _I32),
                       pltpu.VMEM((NPAD // 32,), _F32),
                       pltpu.VMEM((NPAD // 32,), _F32),
                       pltpu.VMEM_SHARED((16 * NPAD,), _F32)],
        compiler_params=_SC_PARAMS,
    )(_deg_body)


def _deg_body(epd, ept, cnt_d, cnt_t, acc_v, ids_v, tmp_v, sum_v, shared):
    c = lax.axis_index("c")
    s = lax.axis_index("s")
    ones = jnp.full((16,), 1.0, _F32)
    epr = E // 16
    colw = NPAD // 32

    def count_into_acc(ep_ref):
        _zero_vmem(acc_v, NZV)

        def chunk(cc, _):
            pltpu.sync_copy(ep_ref.at[pl.ds(s * epr + cc * 2000, 2000)], ids_v)

            def vec(i, _):
                u = ids_v[pl.ds(i * 16, 16)]
                d = jnp.bitwise_and(u >> 16, 0xFFFF)
                plsc.addupdate_scatter(acc_v, [d], ones)
                return 0

            lax.fori_loop(0, 125, vec, 0, unroll=5)
            return 0

        lax.fori_loop(0, 25, chunk, 0)

    @pl.when(c == 0)
    def _():
        count_into_acc(epd)

    @pl.when(c == 1)
    def _():
        count_into_acc(ept)

    pltpu.sync_copy(acc_v, shared.at[pl.ds(s * NPAD, NPAD)])
    plsc.subcore_barrier()

    col0 = s * (NPAD // 16)

    def reduce_half(h, out_ref):
        base = col0 + h * colw
        _zero_vmem(sum_v, colw // 16)

        def row(r, _):
            pltpu.sync_copy(shared.at[pl.ds(r * NPAD + base, colw)], tmp_v)

            def add(j, _):
                sl = pl.ds(j * 16, 16)
                sum_v[sl] = sum_v[sl] + tmp_v[sl]
                return 0

            lax.fori_loop(0, colw // 16, add, 0, unroll=4)
            return 0

        lax.fori_loop(0, 16, row, 0)
        pltpu.sync_copy(sum_v, out_ref.at[pl.ds(base, colw)])

    @pl.when(c == 0)
    def _():
        reduce_half(0, cnt_d)
        reduce_half(1, cnt_d)

    @pl.when(c == 1)
    def _():
        reduce_half(0, cnt_t)
        reduce_half(1, cnt_t)


@functools.lru_cache(maxsize=None)
def _make_agg(fd, ft):
    npass = -(-(fd + ft) // NW)

    @functools.partial(
        pl.kernel,
        out_type=(jax.ShapeDtypeStruct((fd, NPAD), _F32),
                  jax.ShapeDtypeStruct((ft, NPAD), _F32)),
        mesh=_get_mesh(),
        scratch_types=[pltpu.VMEM((NPAD,), _F32),
                       pltpu.VMEM((NPAD,), _F32),
                       pltpu.VMEM((CE,), _I32)],
        compiler_params=_SC_PARAMS,
    )
    def agg(yd, yt, epd, ept, sd, st, y_v, out_v, ids_v):
        c = lax.axis_index("c")
        s = lax.axis_index("s")
        wid = s + 16 * c

        def do_feature(y_ref, ep_ref, out_hbm, f):
            pltpu.sync_copy(y_ref.at[f], y_v)
            _zero_vmem(out_v, NZV)

            def chunk(cc, _):
                pltpu.sync_copy(ep_ref.at[pl.ds(cc * CE, CE)], ids_v)

                @plsc.parallel_loop(0, CE // 16, unroll=8)
                def _(i):
                    u = ids_v[pl.ds(i * 16, 16)]
                    d = jnp.bitwise_and(u >> 16, 0xFFFF)
                    src = jnp.bitwise_and(u, 0xFFFF)
                    g = plsc.load_gather(y_v, [src])
                    plsc.addupdate_scatter(out_v, [d], g)

                return 0

            lax.fori_loop(0, NCH, chunk, 0)
            pltpu.sync_copy(out_v, out_hbm.at[f])

        def passes(p, _):
            fid = p * NW + wid

            @pl.when(fid < fd)
            def _():
                do_feature(yd, epd, sd, fid)

            @pl.when(jnp.logical_and(fid >= fd, fid < fd + ft))
            def _():
                do_feature(yt, ept, st, fid - fd)

            return 0

        lax.fori_loop(0, npass, passes, 0)

    return agg


@functools.lru_cache(maxsize=None)
def _get_pool_kernel():
    return functools.partial(
        pl.kernel,
        out_type=(jax.ShapeDtypeStruct((320, BLKN), _F32),
                  jax.ShapeDtypeStruct((128, BLKN), _F32)),
        mesh=_get_mesh(),
        scratch_types=[pltpu.VMEM((NPAD,), _F32),
                       pltpu.VMEM((BLKN,), _F32),
                       pltpu.VMEM((528,), _I32),
                       pltpu.VMEM((528,), _I32),
                       pltpu.VMEM((528,), _I32),
                       pltpu.VMEM((528,), _I32)],
        compiler_params=_SC_PARAMS,
    )(_pool_body)


def _pool_body(hd, ht, ssd, sed, sst, set_, gd, gt,
               h_v, g_v, sa_d, se_d, sa_t, se_t):
    c = lax.axis_index("c")
    s = lax.axis_index("s")
    wid = s + 16 * c
    pltpu.sync_copy(ssd, sa_d)
    pltpu.sync_copy(sed, se_d)
    pltpu.sync_copy(sst, sa_t)
    pltpu.sync_copy(set_, se_t)
    lanes = lax.iota(_I32, 16)
    ninf = jnp.full((16,), -jnp.inf, _F32)

    def do_feature(h_ref, sa_v, se_v, g_ref, f):
        pltpu.sync_copy(h_ref.at[f], h_v)

        def grp(g, _):
            sv = sa_v[pl.ds(g * 16, 16)]
            ev = se_v[pl.ds(g * 16, 16)]
            res = ninf
            for j in range(16):
                a = jnp.sum(jnp.where(lanes == j, sv, 0))
                b = jnp.sum(jnp.where(lanes == j, ev, 0))
                a0 = jnp.bitwise_and(a, -16)
                nt = (b - a0 + 15) >> 4

                def scanrows(k, m):
                    base = a0 + k * 16
                    v = h_v[pl.ds(base, 16)]
                    lane = base + lanes
                    ok = jnp.logical_and(lane >= a, lane < b)
                    return jnp.maximum(m, jnp.where(ok, v, ninf))

                m = lax.fori_loop(0, nt, scanrows, ninf)
                gmax = jnp.max(m)
                res = jnp.where(lanes == j, gmax, res)
            g_v[pl.ds(g * 16, 16)] = res
            return 0

        lax.fori_loop(0, BLKN // 16, grp, 0)
        pltpu.sync_copy(g_v, g_ref.at[f])

    def passes(p, _):
        fid = p * NW + wid

        @pl.when(fid < 320)
        def _():
            do_feature(hd, sa_d, se_d, gd, fid)

        @pl.when(jnp.logical_and(fid >= 320, fid < 448))
        def _():
            do_feature(ht, sa_t, se_t, gt, fid - 320)

        return 0

    lax.fori_loop(0, 14, passes, 0)


_BF16 = jnp.bfloat16


def _bdot(w, z):
    return jnp.dot(w.astype(_BF16), z.astype(_BF16),
                   preferred_element_type=_F32)


def _k0(xp, dinv_row, w1_t, fp, fo):
    def body(x_ref, d_ref, w_ref, y_ref):
        xt = jnp.transpose(x_ref[...])
        y_ref[...] = d_ref[...] * _bdot(w_ref[...], xt)

    return pl.pallas_call(
        body,
        grid=(NBLK,),
        in_specs=[pl.BlockSpec((BLKN, fp), lambda n: (n, 0)),
                  pl.BlockSpec((1, BLKN), lambda n: (0, n)),
                  pl.BlockSpec((fo, fp), lambda n: (0, 0))],
        out_specs=pl.BlockSpec((fo, BLKN), lambda n: (0, n)),
        out_shape=jax.ShapeDtypeStruct((fo, NPAD), _F32),
    )(xp, dinv_row, w1_t)


def _layer_mid(s_t, y_t, dinv, b_col, wn_t):
    fin = s_t.shape[0]
    fout = wn_t.shape[0]

    def body(s_ref, y_ref, d_ref, b_ref, w_ref, o_ref):
        dv = d_ref[...]
        h = jnp.maximum(dv * (s_ref[...] + y_ref[...]) + b_ref[...], 0.0)
        o_ref[...] = dv * _bdot(w_ref[...], h)

    return pl.pallas_call(
        body,
        grid=(NBLK,),
        in_specs=[pl.BlockSpec((fin, BLKN), lambda n: (0, n)),
                  pl.BlockSpec((fin, BLKN), lambda n: (0, n)),
                  pl.BlockSpec((1, BLKN), lambda n: (0, n)),
                  pl.BlockSpec((fin, 1), lambda n: (0, 0)),
                  pl.BlockSpec((fout, fin), lambda n: (0, 0))],
        out_specs=pl.BlockSpec((fout, BLKN), lambda n: (0, n)),
        out_shape=jax.ShapeDtypeStruct((fout, NPAD), _F32),
    )(s_t, y_t, dinv, b_col, wn_t)


def _layer_fin(s_t, y_t, dinv, b_col):
    fin = s_t.shape[0]

    def body(s_ref, y_ref, d_ref, b_ref, o_ref):
        o_ref[...] = jnp.maximum(
            d_ref[...] * (s_ref[...] + y_ref[...]) + b_ref[...], 0.0)

    return pl.pallas_call(
        body,
        grid=(NBLK,),
        in_specs=[pl.BlockSpec((fin, BLKN), lambda n: (0, n)),
                  pl.BlockSpec((fin, BLKN), lambda n: (0, n)),
                  pl.BlockSpec((1, BLKN), lambda n: (0, n)),
                  pl.BlockSpec((fin, 1), lambda n: (0, 0))],
        out_specs=pl.BlockSpec((fin, BLKN), lambda n: (0, n)),
        out_shape=jax.ShapeDtypeStruct((fin, NPAD), _F32),
    )(s_t, y_t, dinv, b_col)


def _head(gd, gt, w):
    def body(gd_ref, gt_ref, w1d, b1d, w2d, b2d, w1t, b1t, w2t, b2t,
             wca, wcb, bc, w2, b2, wo, bo, o_ref):
        g1d = jnp.maximum(_bdot(w1d[...], gd_ref[...]) + b1d[...], 0.0)
        g2d = _bdot(w2d[...], g1d) + b2d[...]
        g1t = jnp.maximum(_bdot(w1t[...], gt_ref[...]) + b1t[...], 0.0)
        g2t = _bdot(w2t[...], g1t) + b2t[...]
        c1 = jnp.maximum(_bdot(wca[...], g2d)
                         + _bdot(wcb[...], g2t)
                         + bc[...], 0.0)
        c2 = jnp.maximum(_bdot(w2[...], c1)
                         + b2[...], 0.0)
        o_ref[...] = _bdot(wo[...], c2) + bo[...]

    full = lambda a: pl.BlockSpec(a.shape, lambda: tuple(0 for _ in a.shape))
    args = [gd, gt] + w
    return pl.pallas_call(
        body,
        in_specs=[full(a) for a in args],
        out_specs=pl.BlockSpec((1, BLKN), lambda: (0, 0)),
        out_shape=jax.ShapeDtypeStruct((1, BLKN), _F32),
    )(*args)


def _pad_wt(wmat, fin_p, fout_p):
    wt = wmat.T
    return jnp.zeros((fout_p, fin_p), _F32).at[:wt.shape[0], :wt.shape[1]].set(wt)


def _pad_b(b, fp):
    return jnp.zeros((fp, 1), _F32).at[:b.shape[0], 0].set(b)


def kernel(x, edge_index, batch, xt, edge_index_t, batch_t, params):
    p = params

    xp = jnp.zeros((NPAD, 80), _F32).at[:N, :78].set(x)
    xtp = jnp.zeros((NPAD, 32), _F32).at[:N, :30].set(xt)
    epd = jnp.bitwise_or(edge_index[1] << 16, edge_index[0])
    ept = jnp.bitwise_or(edge_index_t[1] << 16, edge_index_t[0])

    segs = jnp.arange(BSEG + 1, dtype=_I32)
    ssd_full = jnp.searchsorted(batch, segs).astype(_I32)
    sst_full = jnp.searchsorted(batch_t, segs).astype(_I32)
    padn = jnp.full((27,), N, _I32)
    ssd = jnp.concatenate([ssd_full[:-1], padn, jnp.zeros((1,), _I32)])[:528]
    sed = jnp.concatenate([ssd_full[1:], padn, jnp.zeros((1,), _I32)])[:528]
    sst = jnp.concatenate([sst_full[:-1], padn, jnp.zeros((1,), _I32)])[:528]
    set_ = jnp.concatenate([sst_full[1:], padn, jnp.zeros((1,), _I32)])[:528]

    cnt_d, cnt_t = _get_deg_kernel()(epd, ept)

    w1d = _pad_wt(p['W1'], 80, 80)
    w2d = _pad_wt(p['W2'], 80, 160)
    w3d = _pad_wt(p['W3'], 160, 320)
    w4d = _pad_wt(p['W4'], 320, 320)
    w1t = _pad_wt(p['Wt1'], 32, 32)
    w2t = _pad_wt(p['Wt2'], 32, 64)
    w3t = _pad_wt(p['Wt3'], 64, 128)
    w4t = _pad_wt(p['Wt4'], 128, 128)
    bd = [_pad_b(p['b1'], 80), _pad_b(p['b2'], 160),
          _pad_b(p['b3'], 320), _pad_b(p['b4'], 320)]
    bt = [_pad_b(p['bt1'], 32), _pad_b(p['bt2'], 64),
          _pad_b(p['bt3'], 128), _pad_b(p['bt4'], 128)]

    dinv_d = lax.rsqrt(cnt_d + 1.0).reshape(1, NPAD)
    dinv_t = lax.rsqrt(cnt_t + 1.0).reshape(1, NPAD)
    y_d = _k0(xp, dinv_d, w1d, 80, 80)
    y_t = _k0(xtp, dinv_t, w1t, 32, 32)

    aggs = [_make_agg(80, 32), _make_agg(160, 64),
            _make_agg(320, 128), _make_agg(320, 128)]
    wnext_d = [w2d, w3d, w4d]
    wnext_t = [w2t, w3t, w4t]
    for i in range(3):
        s_d, s_t = aggs[i](y_d, y_t, epd, ept)
        y_d = _layer_mid(s_d, y_d, dinv_d, bd[i], wnext_d[i])
        y_t = _layer_mid(s_t, y_t, dinv_t, bt[i], wnext_t[i])
    s_d, s_t = aggs[3](y_d, y_t, epd, ept)
    h_d = _layer_fin(s_d, y_d, dinv_d, bd[3])
    h_t = _layer_fin(s_t, y_t, dinv_t, bt[3])

    g_d, g_t = _get_pool_kernel()(h_d, h_t, ssd, sed, sst, set_)

    headw = [_pad_wt(p['fg1W'], 320, 1024), _pad_b(p['fg1b'], 1024),
             _pad_wt(p['fg2W'], 1024, 1280), _pad_b(p['fg2b'], 1280),
             _pad_wt(p['fg1tW'], 128, 1024), _pad_b(p['fg1tb'], 1024),
             _pad_wt(p['fg2tW'], 1024, 1280), _pad_b(p['fg2tb'], 1280),
             _pad_wt(p['fc1W'][:1280], 1280, 1024),
             _pad_wt(p['fc1W'][1280:], 1280, 1024), _pad_b(p['fc1b'], 1024),
             _pad_wt(p['fc2W'], 1024, 512), _pad_b(p['fc2b'], 512),
             _pad_wt(p['outW'], 512, 1), _pad_b(p['outb'], 1)]

    out = _head(g_d, g_t, headw)
    return out[0, :BSEG][:, None]
